# Initial kernel scaffold; baseline (speedup 1.0000x reference)
import jax
import jax.numpy as jnp
from jax.experimental import pallas as pl


def kernel(node_pos, raw_edge_index, batch):
    raise NotImplementedError("write your pallas kernel here")


# speedup vs baseline: 115.6844x; 115.6844x over previous
import functools

import jax
import jax.numpy as jnp
from jax import lax
from jax.experimental import pallas as pl
from jax.experimental.pallas import tpu as pltpu
from jax.experimental.pallas import tpu_sc as plsc

_N_NODES = 100000
_N_EDGES = 6400000
_N_GRAPHS = 128

_NC, _NS, _L = 2, 16, 16
_NW = _NC * _NS
_CHUNK = 1024
_GB = 128
_NG = _CHUNK // _GB
_NCHUNKS = _N_EDGES // _CHUNK
_CPW = -(-_NCHUNKS // _NW)


def _sc_body(tx_h, ty_h, tz_h, tb_h, src_hbm, dst_hbm, out_hbm,
             tx, ty, tz, tb, sidx, didx,
             sxb, syb, szb, sbb, dxb, dyb, dzb,
             bins_ev, bins_cnt, sem):
    c = lax.axis_index("c")
    s = lax.axis_index("s")
    w = s * _NC + c

    @pl.when(s == 0)
    def _stage():
        pltpu.sync_copy(tx_h, tx)
        pltpu.sync_copy(ty_h, ty)
        pltpu.sync_copy(tz_h, tz)
        pltpu.sync_copy(tb_h, tb)

    plsc.subcore_barrier()

    zeros = jnp.zeros((_L,), jnp.float32)

    def _zero(i, carry):
        bins_ev[pl.ds(i * _L, _L)] = zeros
        bins_cnt[pl.ds(i * _L, _L)] = zeros
        return carry

    lax.fori_loop(0, _N_GRAPHS, _zero, 0)

    lane = lax.iota(jnp.int32, _L)
    ones = jnp.ones((_L,), jnp.float32)

    def _chunk(i, carry):
        cid = w + i * _NW

        @pl.when(cid < _NCHUNKS)
        def _do():
            row0 = cid * _NG
            pltpu.sync_copy(src_hbm.at[pl.ds(row0, _NG)], sidx)
            pltpu.sync_copy(dst_hbm.at[pl.ds(row0, _NG)], didx)
            copies = []
            for j in range(_NG):
                sl = pl.ds(j * _GB, _GB)
                si = sidx.at[j]
                di = didx.at[j]
                copies.append(pltpu.async_copy(tx.at[si], sxb.at[sl], sem))
                copies.append(pltpu.async_copy(ty.at[si], syb.at[sl], sem))
                copies.append(pltpu.async_copy(tz.at[si], szb.at[sl], sem))
                copies.append(pltpu.async_copy(tb.at[si], sbb.at[sl], sem))
                copies.append(pltpu.async_copy(tx.at[di], dxb.at[sl], sem))
                copies.append(pltpu.async_copy(ty.at[di], dyb.at[sl], sem))
                copies.append(pltpu.async_copy(tz.at[di], dzb.at[sl], sem))
            for cp in copies:
                cp.wait()

            def _vec(k, carry2):
                sl = pl.ds(k * _L, _L)
                dx = dxb[sl] - sxb[sl]
                dy = dyb[sl] - syb[sl]
                dz = dzb[sl] - szb[sl]
                ss = dx * dx + dy * dy + dz * dz
                sc = jnp.maximum(ss, 1e-30)
                yi = 0x5F3759DF - (lax.bitcast_convert_type(sc, jnp.int32)
                                   >> 1)
                y = lax.bitcast_convert_type(yi, jnp.float32)
                hs = 0.5 * sc
                y = y * (1.5 - hs * y * y)
                y = y * (1.5 - hs * y * y)
                y = y * (1.5 - hs * y * y)
                eu = ss * y
                em = eu - 1.0
                ev = em * em
                gi = sbb[sl] * _L + lane
                plsc.addupdate_scatter(bins_ev, [gi], ev)
                plsc.addupdate_scatter(bins_cnt, [gi], ones)
                return carry2

            lax.fori_loop(0, _CHUNK // _L, _vec, 0)

        return carry

    lax.fori_loop(0, _CPW, _chunk, 0)

    pltpu.sync_copy(bins_ev, out_hbm.at[w, 0])
    pltpu.sync_copy(bins_cnt, out_hbm.at[w, 1])


def _finish_body(ev_ref, cnt_ref, o_ref):
    ev = jnp.sum(ev_ref[...], axis=0)
    cnt = jnp.sum(cnt_ref[...], axis=0)
    evg = jnp.sum(ev, axis=1, keepdims=True)
    cg = jnp.sum(cnt, axis=1, keepdims=True)
    gv = jnp.where(cg > 0, evg / jnp.maximum(cg, 1.0), 0.0)
    o_ref[...] = (jnp.sum(gv) / jnp.float32(_N_GRAPHS)).reshape(1, 1)


@jax.jit
def _run(tx, ty, tz, tb, src2d, dst2d):
    mesh = plsc.VectorSubcoreMesh(core_axis_name="c", subcore_axis_name="s")
    partials = pl.kernel(
        _sc_body,
        out_type=jax.ShapeDtypeStruct((_NW, 2, _N_GRAPHS * _L), jnp.float32),
        mesh=mesh,
        compiler_params=pltpu.CompilerParams(needs_layout_passes=False),
        scratch_types=[
            pltpu.VMEM_SHARED((_N_NODES,), jnp.float32),
            pltpu.VMEM_SHARED((_N_NODES,), jnp.float32),
            pltpu.VMEM_SHARED((_N_NODES,), jnp.float32),
            pltpu.VMEM_SHARED((_N_NODES,), jnp.int32),
            pltpu.VMEM((_NG, _GB), jnp.int32),
            pltpu.VMEM((_NG, _GB), jnp.int32),
            pltpu.VMEM((_CHUNK,), jnp.float32),
            pltpu.VMEM((_CHUNK,), jnp.float32),
            pltpu.VMEM((_CHUNK,), jnp.float32),
            pltpu.VMEM((_CHUNK,), jnp.int32),
            pltpu.VMEM((_CHUNK,), jnp.float32),
            pltpu.VMEM((_CHUNK,), jnp.float32),
            pltpu.VMEM((_CHUNK,), jnp.float32),
            pltpu.VMEM((_N_GRAPHS * _L,), jnp.float32),
            pltpu.VMEM((_N_GRAPHS * _L,), jnp.float32),
            pltpu.SemaphoreType.DMA,
        ],
    )(tx, ty, tz, tb, src2d, dst2d)

    ev_part = partials[:, 0, :].reshape(_NW, _N_GRAPHS, _L)
    cnt_part = partials[:, 1, :].reshape(_NW, _N_GRAPHS, _L)
    res = pl.pallas_call(
        _finish_body,
        out_shape=jax.ShapeDtypeStruct((1, 1), jnp.float32),
    )(ev_part, cnt_part)
    return res[0, 0]


def kernel(node_pos, raw_edge_index, batch):
    ei = raw_edge_index.astype(jnp.int32)
    src2d = ei[0].reshape(_N_EDGES // _GB, _GB)
    dst2d = ei[1].reshape(_N_EDGES // _GB, _GB)
    pos = node_pos.astype(jnp.float32)
    return _run(pos[:, 0], pos[:, 1], pos[:, 2], batch.astype(jnp.int32),
                src2d, dst2d)
